# Initial kernel scaffold; baseline (speedup 1.0000x reference)
import jax
import jax.numpy as jnp
from jax.experimental import pallas as pl


def kernel(x, edge_index, W1, b1, W2, b2, W3, b3):
    raise NotImplementedError("write your pallas kernel here")


# speedup vs baseline: 7.6229x; 7.6229x over previous
import functools

import jax
import jax.numpy as jnp
from jax import lax
from jax.experimental import pallas as pl
from jax.experimental.pallas import tpu as pltpu
from jax.experimental.pallas import tpu_sc as plsc

N_NODES = 10000
N_EDGES = 320000
N_PAD = 10240

NC, NS = 2, 16
NW = NC * NS
E_PER_W = N_EDGES // NW
CHUNK = 80
N_CHUNKS = E_PER_W // CHUNK
ROWS_PER_TILE = N_PAD // NS
ZROWS = 128

_mesh = lambda: plsc.VectorSubcoreMesh(core_axis_name="c", subcore_axis_name="s")


def _zero_fill(buf, rows, cols):
    per_row = cols // 16

    @pl.loop(0, rows * per_row)
    def _(i):
        buf[i // per_row, pl.ds((i % per_row) * 16, 16)] = jnp.zeros((16,), jnp.float32)


def _zero_acc(acc, zbuf, s):
    @pl.loop(0, ROWS_PER_TILE // ZROWS)
    def _(i):
        pltpu.sync_copy(zbuf, acc.at[pl.ds(s * ROWS_PER_TILE + i * ZROWS, ZROWS)])


def _copy_out(acc, out, c, s):
    @pl.loop(0, ROWS_PER_TILE // ZROWS)
    def _(i):
        r0 = s * ROWS_PER_TILE + i * ZROWS
        pltpu.sync_copy(acc.at[pl.ds(r0, ZROWS)], out.at[pl.ds(c * N_PAD + r0, ZROWS)])


@functools.lru_cache(maxsize=None)
def _make_segsum(d):
    @functools.partial(
        pl.kernel,
        out_type=jax.ShapeDtypeStruct((NC * N_PAD, d), jnp.float32),
        mesh=_mesh(),
        compiler_params=pltpu.CompilerParams(use_tc_tiling_on_sc=(d % 128 == 0)),
        scratch_types=[
            pltpu.VMEM_SHARED((N_PAD, d), jnp.float32),
            pltpu.VMEM((CHUNK,), jnp.int32),
            pltpu.VMEM((CHUNK,), jnp.int32),
            pltpu.VMEM((CHUNK, d), jnp.float32),
            pltpu.VMEM((ZROWS, d), jnp.float32),
            pltpu.SemaphoreType.DMA,
        ],
    )
    def segsum(u_hbm, src_hbm, dst_hbm, out_hbm, acc, sidx, didx, rows, zbuf, sem):
        c = lax.axis_index("c")
        s = lax.axis_index("s")
        w = s * NC + c
        _zero_fill(zbuf, ZROWS, d)
        _zero_acc(acc, zbuf, s)
        plsc.subcore_barrier()

        @pl.loop(0, N_CHUNKS)
        def _(i):
            base = w * E_PER_W + i * CHUNK
            pltpu.sync_copy(src_hbm.at[pl.ds(base, CHUNK)], sidx)
            pltpu.sync_copy(dst_hbm.at[pl.ds(base, CHUNK)], didx)
            pltpu.async_copy(u_hbm.at[sidx], rows, sem).wait()
            pltpu.sync_copy(rows, acc.at[didx], add=True)

        plsc.subcore_barrier()
        _copy_out(acc, out_hbm, c, s)

    return segsum


DEG_W = 16


@functools.lru_cache(maxsize=None)
def _make_deg_count():
    return functools.partial(
        pl.kernel,
        out_type=jax.ShapeDtypeStruct((NC * N_PAD, DEG_W), jnp.float32),
        mesh=_mesh(),
        compiler_params=pltpu.CompilerParams(use_tc_tiling_on_sc=False),
        scratch_types=[
            pltpu.VMEM_SHARED((N_PAD, DEG_W), jnp.float32),
            pltpu.VMEM((CHUNK,), jnp.int32),
            pltpu.VMEM((ZROWS, DEG_W), jnp.float32),
            pltpu.VMEM((CHUNK, DEG_W), jnp.float32),
        ],
    )(_deg_count_body)


def _deg_count_body(dst_hbm, out_hbm, acc, didx, zbuf, ones):
    c = lax.axis_index("c")
    s = lax.axis_index("s")
    w = s * NC + c
    _zero_fill(zbuf, ZROWS, DEG_W)
    _zero_acc(acc, zbuf, s)

    per_row = DEG_W // 16

    @pl.loop(0, CHUNK * per_row)
    def _(i):
        ones[i // per_row, pl.ds((i % per_row) * 16, 16)] = jnp.ones((16,), jnp.float32)

    plsc.subcore_barrier()

    @pl.loop(0, N_CHUNKS)
    def _(i):
        base = w * E_PER_W + i * CHUNK
        pltpu.sync_copy(dst_hbm.at[pl.ds(base, CHUNK)], didx)
        pltpu.sync_copy(ones, acc.at[didx], add=True)

    plsc.subcore_barrier()
    _copy_out(acc, out_hbm, c, s)


@functools.lru_cache(maxsize=None)
def _make_decode():
    return functools.partial(
        pl.kernel,
        out_type=jax.ShapeDtypeStruct((N_EDGES,), jnp.float32),
        mesh=_mesh(),
        compiler_params=pltpu.CompilerParams(
            use_tc_tiling_on_sc=False, needs_layout_passes=False),
        scratch_types=[
            pltpu.VMEM((CHUNK,), jnp.int32),
            pltpu.VMEM((CHUNK,), jnp.int32),
            pltpu.VMEM((CHUNK, 64), jnp.float32),
            pltpu.VMEM((CHUNK, 64), jnp.float32),
            pltpu.VMEM((CHUNK,), jnp.float32),
            pltpu.SemaphoreType.DMA,
            pltpu.SemaphoreType.DMA,
        ],
    )(_decode_body)


def _decode_body(z_hbm, src_hbm, dst_hbm, out_hbm, sidx, didx, srows, drows, obuf, sem1, sem2):
    c = lax.axis_index("c")
    s = lax.axis_index("s")
    w = s * NC + c

    @pl.loop(0, N_CHUNKS)
    def _(i):
        base = w * E_PER_W + i * CHUNK
        pltpu.sync_copy(src_hbm.at[pl.ds(base, CHUNK)], sidx)
        pltpu.sync_copy(dst_hbm.at[pl.ds(base, CHUNK)], didx)
        cp1 = pltpu.async_copy(z_hbm.at[sidx], srows, sem1)
        cp2 = pltpu.async_copy(z_hbm.at[didx], drows, sem2)
        cp1.wait()
        cp2.wait()
        for g in range(CHUNK // 16):
            row = jnp.full((16,), g * 16, jnp.int32) + lax.iota(jnp.int32, 16)
            acc16 = jnp.zeros((16,), jnp.float32)
            for k in range(64):
                col = jnp.full((16,), k, jnp.int32)
                sv = plsc.load_gather(srows, [row, col])
                dv = plsc.load_gather(drows, [row, col])
                acc16 = acc16 + sv * dv
            prob = 1.0 / (1.0 + jnp.exp(-acc16))
            obuf[pl.ds(g * 16, 16)] = prob
        pltpu.sync_copy(obuf, out_hbm.at[pl.ds(base, CHUNK)])


TC_R = 512
TC_GRID = N_PAD // TC_R


def _row_spec(d):
    return pl.BlockSpec((TC_R, d), lambda i: (i, 0))


def _part_specs(d):
    return (
        pl.BlockSpec((TC_R, d), lambda i: (i, 0)),
        pl.BlockSpec((TC_R, d), lambda i: (i + TC_GRID, 0)),
    )


def _full_spec(shape):
    return pl.BlockSpec(shape, lambda i: tuple(0 for _ in shape))


def _prep_body(cnt_a, cnt_b, x, dinv_out, u1_out):
    deg = cnt_a[:, 0:1] + cnt_b[:, 0:1] + 1.0
    dinv = jnp.broadcast_to(lax.rsqrt(deg), (TC_R, 128))
    dinv_out[...] = dinv
    u1_out[...] = dinv * x[...]


def _layer1_body(s1a, s1b, u1, dinv, w1, b1, w2, u2_out):
    dv = dinv[...]
    b = dv * (s1a[...] + s1b[...] + u1[...])
    h1 = jnp.maximum(jnp.dot(b, w1[...], preferred_element_type=jnp.float32) + b1[...], 0.0)
    y2 = jnp.dot(h1, w2[...], preferred_element_type=jnp.float32)
    u2_out[...] = dv * y2


def _layer2_body(s2a, s2b, u2, dinv, b2, w3, u3_out):
    dv = dinv[...]
    h2 = jnp.maximum(dv * (s2a[...] + s2b[...] + u2[...]) + b2[...], 0.0)
    y3 = jnp.dot(h2, w3[...], preferred_element_type=jnp.float32)
    u3_out[...] = dv[:, 0:64] * y3


def _layer3_body(s3a, s3b, u3, dinv, b3, z_out):
    dv = dinv[:, 0:64]
    z_out[...] = jnp.maximum(dv * (s3a[...] + s3b[...] + u3[...]) + b3[...], 0.0)


def kernel(x, edge_index, W1, b1, W2, b2, W3, b3):
    ei = edge_index.astype(jnp.int32)
    src, dst = ei[0], ei[1]

    cnt = _make_deg_count()(dst)

    dinv, u1 = pl.pallas_call(
        _prep_body,
        grid=(TC_GRID,),
        in_specs=[*_part_specs(DEG_W), _row_spec(128)],
        out_specs=[_row_spec(128), _row_spec(128)],
        out_shape=[
            jax.ShapeDtypeStruct((N_NODES, 128), jnp.float32),
            jax.ShapeDtypeStruct((N_NODES, 128), jnp.float32),
        ],
    )(cnt, cnt, x)

    s1 = _make_segsum(128)(u1, src, dst)
    u2 = pl.pallas_call(
        _layer1_body,
        grid=(TC_GRID,),
        in_specs=[
            *_part_specs(128), _row_spec(128), _row_spec(128),
            _full_spec((128, 256)), _full_spec((1, 256)), _full_spec((256, 128)),
        ],
        out_specs=_row_spec(128),
        out_shape=jax.ShapeDtypeStruct((N_NODES, 128), jnp.float32),
    )(s1, s1, u1, dinv, W1, b1.reshape(1, -1), W2)

    s2 = _make_segsum(128)(u2, src, dst)
    u3 = pl.pallas_call(
        _layer2_body,
        grid=(TC_GRID,),
        in_specs=[
            *_part_specs(128), _row_spec(128), _row_spec(128),
            _full_spec((1, 128)), _full_spec((128, 64)),
        ],
        out_specs=_row_spec(64),
        out_shape=jax.ShapeDtypeStruct((N_NODES, 64), jnp.float32),
    )(s2, s2, u2, dinv, b2.reshape(1, -1), W3)

    s3 = _make_segsum(64)(u3, src, dst)
    z = pl.pallas_call(
        _layer3_body,
        grid=(TC_GRID,),
        in_specs=[
            *_part_specs(64), _row_spec(64), _row_spec(128), _full_spec((1, 64)),
        ],
        out_specs=_row_spec(64),
        out_shape=jax.ShapeDtypeStruct((N_NODES, 64), jnp.float32),
    )(s3, s3, u3, dinv, b3.reshape(1, -1))

    return _make_decode()(z, src, dst)


# speedup vs baseline: 13.6041x; 1.7846x over previous
import functools

import jax
import jax.numpy as jnp
from jax import lax
from jax.experimental import pallas as pl
from jax.experimental.pallas import tpu as pltpu
from jax.experimental.pallas import tpu_sc as plsc

N_NODES = 10000
N_EDGES = 320000
N_PAD = 10240

NC, NS = 2, 16
NW = NC * NS
E_PER_W = N_EDGES // NW
CHUNK = 40
N_CHUNKS = E_PER_W // CHUNK
ROWS_PER_TILE = N_PAD // NS
ZROWS = 128

_mesh = lambda: plsc.VectorSubcoreMesh(core_axis_name="c", subcore_axis_name="s")


def _zero_fill(buf, rows, cols):
    per_row = cols // 16

    @pl.loop(0, rows * per_row)
    def _(i):
        buf[i // per_row, pl.ds((i % per_row) * 16, 16)] = jnp.zeros((16,), jnp.float32)


def _zero_acc(acc, zbuf, s):
    @pl.loop(0, ROWS_PER_TILE // ZROWS)
    def _(i):
        pltpu.sync_copy(zbuf, acc.at[pl.ds(s * ROWS_PER_TILE + i * ZROWS, ZROWS)])


def _copy_out(acc, out, c, s):
    @pl.loop(0, ROWS_PER_TILE // ZROWS)
    def _(i):
        r0 = s * ROWS_PER_TILE + i * ZROWS
        pltpu.sync_copy(acc.at[pl.ds(r0, ZROWS)], out.at[pl.ds(c * N_PAD + r0, ZROWS)])


NBUF = 5
ROWS_PER_W = E_PER_W // CHUNK


@functools.lru_cache(maxsize=None)
def _make_segsum(d):
    @functools.partial(
        pl.kernel,
        out_type=jax.ShapeDtypeStruct((NC * N_PAD, d), jnp.float32),
        mesh=_mesh(),
        compiler_params=pltpu.CompilerParams(use_tc_tiling_on_sc=(d % 128 == 0)),
        scratch_types=[
            pltpu.VMEM_SHARED((N_PAD, d), jnp.float32),
            pltpu.VMEM((E_PER_W,), jnp.int32),
            pltpu.VMEM((E_PER_W,), jnp.int32),
            [pltpu.VMEM((CHUNK, d), jnp.float32) for _ in range(NBUF)],
            [pltpu.SemaphoreType.DMA for _ in range(NBUF)],
            [pltpu.SemaphoreType.DMA for _ in range(NBUF)],
        ],
    )
    def segsum(u_hbm, src_hbm, dst_hbm, out_hbm, acc, sidx, didx, rows,
               gsem, ssem):
        c = lax.axis_index("c")
        s = lax.axis_index("s")
        w = s * NC + c
        _zero_fill(rows[0], CHUNK, d)

        @pl.loop(0, ROWS_PER_TILE // CHUNK)
        def _(i):
            pltpu.sync_copy(rows[0], acc.at[pl.ds(s * ROWS_PER_TILE + i * CHUNK, CHUNK)])

        e0 = w * E_PER_W
        pltpu.sync_copy(src_hbm.at[pl.ds(e0, E_PER_W)], sidx)
        pltpu.sync_copy(dst_hbm.at[pl.ds(e0, E_PER_W)], didx)
        plsc.subcore_barrier()

        def gstart(b, i):
            pltpu.async_copy(u_hbm.at[sidx.at[pl.ds(i * CHUNK, CHUNK)]], rows[b], gsem[b])

        def gwait(b):
            pltpu.make_async_copy(u_hbm.at[sidx.at[pl.ds(0, CHUNK)]], rows[b], gsem[b]).wait()

        def sstart(b, i):
            pltpu.async_copy(rows[b], acc.at[didx.at[pl.ds(i * CHUNK, CHUNK)]], ssem[b], add=True)

        def swait(b):
            pltpu.make_async_copy(rows[b], acc.at[didx.at[pl.ds(0, CHUNK)]], ssem[b]).wait()

        for b in range(NBUF):
            gstart(b, b)

        @pl.loop(0, N_CHUNKS // NBUF)
        def _(g):
            for b in range(NBUF):
                i = g * NBUF + b
                gwait(b)
                sstart(b, i)
                pb = (b - 1) % NBUF
                j = i + NBUF - 1

                @pl.when(jnp.logical_and(i >= 1, j < N_CHUNKS))
                def _():
                    swait(pb)
                    gstart(pb, j)

        for b in range(NBUF):
            swait(b)
        plsc.subcore_barrier()
        _copy_out(acc, out_hbm, c, s)

    return segsum


DEG_W = 16


@functools.lru_cache(maxsize=None)
def _make_deg_count():
    return functools.partial(
        pl.kernel,
        out_type=jax.ShapeDtypeStruct((NC * N_PAD, DEG_W), jnp.float32),
        mesh=_mesh(),
        compiler_params=pltpu.CompilerParams(use_tc_tiling_on_sc=False),
        scratch_types=[
            pltpu.VMEM_SHARED((N_PAD, DEG_W), jnp.float32),
            pltpu.VMEM((E_PER_W,), jnp.int32),
            pltpu.VMEM((ZROWS, DEG_W), jnp.float32),
            pltpu.VMEM((CHUNK, DEG_W), jnp.float32),
            [pltpu.SemaphoreType.DMA for _ in range(NBUF)],
        ],
    )(_deg_count_body)


def _deg_count_body(dst_hbm, out_hbm, acc, didx, zbuf, ones, ssem):
    c = lax.axis_index("c")
    s = lax.axis_index("s")
    w = s * NC + c
    _zero_fill(zbuf, ZROWS, DEG_W)
    _zero_acc(acc, zbuf, s)

    per_row = DEG_W // 16

    @pl.loop(0, CHUNK * per_row)
    def _(i):
        ones[i // per_row, pl.ds((i % per_row) * 16, 16)] = jnp.ones((16,), jnp.float32)

    pltpu.sync_copy(dst_hbm.at[pl.ds(w * E_PER_W, E_PER_W)], didx)
    plsc.subcore_barrier()

    def sstart(b, i):
        pltpu.async_copy(ones, acc.at[didx.at[pl.ds(i * CHUNK, CHUNK)]], ssem[b], add=True)

    def swait(b):
        pltpu.make_async_copy(ones, acc.at[didx.at[pl.ds(0, CHUNK)]], ssem[b]).wait()

    for b in range(NBUF):
        sstart(b, b)

    @pl.loop(0, N_CHUNKS // NBUF)
    def _(g):
        for b in range(NBUF):
            j = g * NBUF + b + NBUF

            @pl.when(j < N_CHUNKS)
            def _():
                swait(b)
                sstart(b, j)

    for b in range(NBUF):
        swait(b)
    plsc.subcore_barrier()
    _copy_out(acc, out_hbm, c, s)


DCHUNK = 80
DNBUF = 4
DN_CHUNKS = E_PER_W // DCHUNK


@functools.lru_cache(maxsize=None)
def _make_decode():
    return functools.partial(
        pl.kernel,
        out_type=jax.ShapeDtypeStruct((N_EDGES,), jnp.float32),
        mesh=_mesh(),
        compiler_params=pltpu.CompilerParams(
            use_tc_tiling_on_sc=False, needs_layout_passes=False),
        scratch_types=[
            pltpu.VMEM((E_PER_W,), jnp.int32),
            pltpu.VMEM((E_PER_W,), jnp.int32),
            [pltpu.VMEM((DCHUNK, 64), jnp.float32) for _ in range(DNBUF)],
            [pltpu.VMEM((DCHUNK, 64), jnp.float32) for _ in range(DNBUF)],
            pltpu.VMEM((E_PER_W,), jnp.float32),
            [pltpu.SemaphoreType.DMA for _ in range(DNBUF)],
            [pltpu.SemaphoreType.DMA for _ in range(DNBUF)],
        ],
    )(_decode_body)


def _decode_body(z_hbm, src_hbm, dst_hbm, out_hbm, sidx, didx, srows, drows,
                 obuf, ssem, dsem):
    c = lax.axis_index("c")
    s = lax.axis_index("s")
    w = s * NC + c
    e0 = w * E_PER_W
    pltpu.sync_copy(src_hbm.at[pl.ds(e0, E_PER_W)], sidx)
    pltpu.sync_copy(dst_hbm.at[pl.ds(e0, E_PER_W)], didx)

    def gstart(b, i):
        pltpu.async_copy(z_hbm.at[sidx.at[pl.ds(i * DCHUNK, DCHUNK)]], srows[b], ssem[b])
        pltpu.async_copy(z_hbm.at[didx.at[pl.ds(i * DCHUNK, DCHUNK)]], drows[b], dsem[b])

    def gwait(b):
        pltpu.make_async_copy(z_hbm.at[sidx.at[pl.ds(0, DCHUNK)]], srows[b], ssem[b]).wait()
        pltpu.make_async_copy(z_hbm.at[didx.at[pl.ds(0, DCHUNK)]], drows[b], dsem[b]).wait()

    def slot(b, i, refill):
        gwait(b)
        for gg in range(DCHUNK // 16):
            row = jnp.full((16,), gg * 16, jnp.int32) + lax.iota(jnp.int32, 16)
            acc16 = jnp.zeros((16,), jnp.float32)
            for k in range(64):
                col = jnp.full((16,), k, jnp.int32)
                sv = plsc.load_gather(srows[b], [row, col])
                dv = plsc.load_gather(drows[b], [row, col])
                acc16 = acc16 + sv * dv
            prob = 1.0 / (1.0 + jnp.exp(-acc16))
            obuf[pl.ds(i * DCHUNK + gg * 16, 16)] = prob
        if refill:
            j = i + DNBUF

            @pl.when(j < DN_CHUNKS)
            def _():
                gstart(b, j)

    for b in range(DNBUF):
        gstart(b, b)

    @pl.loop(0, DN_CHUNKS // DNBUF)
    def _(g):
        for b in range(DNBUF):
            slot(b, g * DNBUF + b, True)

    slot(0, DN_CHUNKS - 1, False)
    pltpu.sync_copy(obuf, out_hbm.at[pl.ds(w * E_PER_W, E_PER_W)])


TC_R = 512
TC_GRID = N_PAD // TC_R


def _row_spec(d):
    return pl.BlockSpec((TC_R, d), lambda i: (i, 0))


def _part_specs(d):
    return (
        pl.BlockSpec((TC_R, d), lambda i: (i, 0)),
        pl.BlockSpec((TC_R, d), lambda i: (i + TC_GRID, 0)),
    )


def _full_spec(shape):
    return pl.BlockSpec(shape, lambda i: tuple(0 for _ in shape))


def _prep_body(cnt_a, cnt_b, x, dinv_out, u1_out):
    deg = cnt_a[:, 0:1] + cnt_b[:, 0:1] + 1.0
    dinv = jnp.broadcast_to(lax.rsqrt(deg), (TC_R, 128))
    dinv_out[...] = dinv
    u1_out[...] = dinv * x[...]


def _layer1_body(s1a, s1b, u1, dinv, w1, b1, w2, u2_out):
    dv = dinv[...]
    b = dv * (s1a[...] + s1b[...] + u1[...])
    h1 = jnp.maximum(jnp.dot(b, w1[...], preferred_element_type=jnp.float32) + b1[...], 0.0)
    y2 = jnp.dot(h1, w2[...], preferred_element_type=jnp.float32)
    u2_out[...] = dv * y2


def _layer2_body(s2a, s2b, u2, dinv, b2, w3, u3_out):
    dv = dinv[...]
    h2 = jnp.maximum(dv * (s2a[...] + s2b[...] + u2[...]) + b2[...], 0.0)
    y3 = jnp.dot(h2, w3[...], preferred_element_type=jnp.float32)
    u3_out[...] = dv[:, 0:64] * y3


def _layer3_body(s3a, s3b, u3, dinv, b3, z_out):
    dv = dinv[:, 0:64]
    z_out[...] = jnp.maximum(dv * (s3a[...] + s3b[...] + u3[...]) + b3[...], 0.0)


def kernel(x, edge_index, W1, b1, W2, b2, W3, b3):
    ei = edge_index.astype(jnp.int32)
    src, dst = ei[0], ei[1]

    cnt = _make_deg_count()(dst)

    dinv, u1 = pl.pallas_call(
        _prep_body,
        grid=(TC_GRID,),
        in_specs=[*_part_specs(DEG_W), _row_spec(128)],
        out_specs=[_row_spec(128), _row_spec(128)],
        out_shape=[
            jax.ShapeDtypeStruct((N_NODES, 128), jnp.float32),
            jax.ShapeDtypeStruct((N_NODES, 128), jnp.float32),
        ],
    )(cnt, cnt, x)

    s1 = _make_segsum(128)(u1, src, dst)
    u2 = pl.pallas_call(
        _layer1_body,
        grid=(TC_GRID,),
        in_specs=[
            *_part_specs(128), _row_spec(128), _row_spec(128),
            _full_spec((128, 256)), _full_spec((1, 256)), _full_spec((256, 128)),
        ],
        out_specs=_row_spec(128),
        out_shape=jax.ShapeDtypeStruct((N_NODES, 128), jnp.float32),
    )(s1, s1, u1, dinv, W1, b1.reshape(1, -1), W2)

    s2 = _make_segsum(128)(u2, src, dst)
    u3 = pl.pallas_call(
        _layer2_body,
        grid=(TC_GRID,),
        in_specs=[
            *_part_specs(128), _row_spec(128), _row_spec(128),
            _full_spec((1, 128)), _full_spec((128, 64)),
        ],
        out_specs=_row_spec(64),
        out_shape=jax.ShapeDtypeStruct((N_NODES, 64), jnp.float32),
    )(s2, s2, u2, dinv, b2.reshape(1, -1), W3)

    s3 = _make_segsum(64)(u3, src, dst)
    z = pl.pallas_call(
        _layer3_body,
        grid=(TC_GRID,),
        in_specs=[
            *_part_specs(64), _row_spec(64), _row_spec(128), _full_spec((1, 64)),
        ],
        out_specs=_row_spec(64),
        out_shape=jax.ShapeDtypeStruct((N_NODES, 64), jnp.float32),
    )(s3, s3, u3, dinv, b3.reshape(1, -1))

    return _make_decode()(z, src, dst)


# speedup vs baseline: 20.5587x; 1.5112x over previous
import functools

import jax
import jax.numpy as jnp
from jax import lax
from jax.experimental import pallas as pl
from jax.experimental.pallas import tpu as pltpu
from jax.experimental.pallas import tpu_sc as plsc

N_NODES = 10000
N_EDGES = 320000
N_PAD = 10240

NC, NS = 2, 16
NW = NC * NS
E_PER_W = N_EDGES // NW
CHUNK = 40
N_CHUNKS = E_PER_W // CHUNK
ROWS_PER_TILE = N_PAD // NS
ZROWS = 128

_mesh = lambda: plsc.VectorSubcoreMesh(core_axis_name="c", subcore_axis_name="s")


def _zero_fill(buf, rows, cols):
    per_row = cols // 16

    @pl.loop(0, rows * per_row)
    def _(i):
        buf[i // per_row, pl.ds((i % per_row) * 16, 16)] = jnp.zeros((16,), jnp.float32)


def _zero_acc(acc, zbuf, s):
    @pl.loop(0, ROWS_PER_TILE // ZROWS)
    def _(i):
        pltpu.sync_copy(zbuf, acc.at[pl.ds(s * ROWS_PER_TILE + i * ZROWS, ZROWS)])


def _copy_out(acc, out, c, s):
    @pl.loop(0, ROWS_PER_TILE // ZROWS)
    def _(i):
        r0 = s * ROWS_PER_TILE + i * ZROWS
        pltpu.sync_copy(acc.at[pl.ds(r0, ZROWS)], out.at[pl.ds(c * N_PAD + r0, ZROWS)])


NBUF = 5
ROWS_PER_W = E_PER_W // CHUNK


@functools.lru_cache(maxsize=None)
def _make_segsum(d):
    @functools.partial(
        pl.kernel,
        out_type=jax.ShapeDtypeStruct((NC * N_PAD, d), jnp.float32),
        mesh=_mesh(),
        compiler_params=pltpu.CompilerParams(use_tc_tiling_on_sc=(d % 128 == 0)),
        scratch_types=[
            pltpu.VMEM_SHARED((N_PAD, d), jnp.float32),
            pltpu.VMEM((E_PER_W,), jnp.int32),
            pltpu.VMEM((E_PER_W,), jnp.int32),
            [pltpu.VMEM((CHUNK, d), jnp.float32) for _ in range(NBUF)],
            [pltpu.SemaphoreType.DMA for _ in range(NBUF)],
            [pltpu.SemaphoreType.DMA for _ in range(NBUF)],
        ],
    )
    def segsum(u_hbm, src_hbm, dst_hbm, out_hbm, acc, sidx, didx, rows,
               gsem, ssem):
        c = lax.axis_index("c")
        s = lax.axis_index("s")
        w = s * NC + c
        _zero_fill(rows[0], CHUNK, d)

        @pl.loop(0, ROWS_PER_TILE // CHUNK)
        def _(i):
            pltpu.sync_copy(rows[0], acc.at[pl.ds(s * ROWS_PER_TILE + i * CHUNK, CHUNK)])

        e0 = w * E_PER_W
        pltpu.sync_copy(src_hbm.at[pl.ds(e0, E_PER_W)], sidx)
        pltpu.sync_copy(dst_hbm.at[pl.ds(e0, E_PER_W)], didx)
        plsc.subcore_barrier()

        def gstart(b, i):
            pltpu.async_copy(u_hbm.at[sidx.at[pl.ds(i * CHUNK, CHUNK)]], rows[b], gsem[b])

        def gwait(b):
            pltpu.make_async_copy(u_hbm.at[sidx.at[pl.ds(0, CHUNK)]], rows[b], gsem[b]).wait()

        def sstart(b, i):
            pltpu.async_copy(rows[b], acc.at[didx.at[pl.ds(i * CHUNK, CHUNK)]], ssem[b], add=True)

        def swait(b):
            pltpu.make_async_copy(rows[b], acc.at[didx.at[pl.ds(0, CHUNK)]], ssem[b]).wait()

        for b in range(NBUF):
            gstart(b, b)

        @pl.loop(0, N_CHUNKS // NBUF)
        def _(g):
            for b in range(NBUF):
                i = g * NBUF + b
                gwait(b)
                sstart(b, i)
                pb = (b - 1) % NBUF
                j = i + NBUF - 1

                @pl.when(jnp.logical_and(i >= 1, j < N_CHUNKS))
                def _():
                    swait(pb)
                    gstart(pb, j)

        for b in range(NBUF):
            swait(b)
        plsc.subcore_barrier()
        _copy_out(acc, out_hbm, c, s)

    return segsum


DEG_W = 16


@functools.lru_cache(maxsize=None)
def _make_deg_count():
    return functools.partial(
        pl.kernel,
        out_type=jax.ShapeDtypeStruct((NC * N_PAD, DEG_W), jnp.float32),
        mesh=_mesh(),
        compiler_params=pltpu.CompilerParams(use_tc_tiling_on_sc=False),
        scratch_types=[
            pltpu.VMEM_SHARED((N_PAD, DEG_W), jnp.float32),
            pltpu.VMEM((E_PER_W,), jnp.int32),
            pltpu.VMEM((ZROWS, DEG_W), jnp.float32),
            pltpu.VMEM((CHUNK, DEG_W), jnp.float32),
            [pltpu.SemaphoreType.DMA for _ in range(NBUF)],
        ],
    )(_deg_count_body)


def _deg_count_body(dst_hbm, out_hbm, acc, didx, zbuf, ones, ssem):
    c = lax.axis_index("c")
    s = lax.axis_index("s")
    w = s * NC + c
    _zero_fill(zbuf, ZROWS, DEG_W)
    _zero_acc(acc, zbuf, s)

    per_row = DEG_W // 16

    @pl.loop(0, CHUNK * per_row)
    def _(i):
        ones[i // per_row, pl.ds((i % per_row) * 16, 16)] = jnp.ones((16,), jnp.float32)

    pltpu.sync_copy(dst_hbm.at[pl.ds(w * E_PER_W, E_PER_W)], didx)
    plsc.subcore_barrier()

    def sstart(b, i):
        pltpu.async_copy(ones, acc.at[didx.at[pl.ds(i * CHUNK, CHUNK)]], ssem[b], add=True)

    def swait(b):
        pltpu.make_async_copy(ones, acc.at[didx.at[pl.ds(0, CHUNK)]], ssem[b]).wait()

    for b in range(NBUF):
        sstart(b, b)

    @pl.loop(0, N_CHUNKS // NBUF)
    def _(g):
        for b in range(NBUF):
            j = g * NBUF + b + NBUF

            @pl.when(j < N_CHUNKS)
            def _():
                swait(b)
                sstart(b, j)

    for b in range(NBUF):
        swait(b)
    plsc.subcore_barrier()
    _copy_out(acc, out_hbm, c, s)


DCHUNK = 80
DNBUF = 4
DN_CHUNKS = E_PER_W // DCHUNK


@functools.lru_cache(maxsize=None)
def _make_decode():
    return functools.partial(
        pl.kernel,
        out_type=jax.ShapeDtypeStruct((N_EDGES,), jnp.float32),
        mesh=_mesh(),
        compiler_params=pltpu.CompilerParams(
            use_tc_tiling_on_sc=False, needs_layout_passes=False),
        scratch_types=[
            pltpu.VMEM((E_PER_W,), jnp.int32),
            pltpu.VMEM((E_PER_W,), jnp.int32),
            [pltpu.VMEM((DCHUNK, 64), jnp.float32) for _ in range(DNBUF)],
            [pltpu.VMEM((DCHUNK, 64), jnp.float32) for _ in range(DNBUF)],
            pltpu.VMEM((E_PER_W,), jnp.float32),
            [pltpu.SemaphoreType.DMA for _ in range(DNBUF)],
            [pltpu.SemaphoreType.DMA for _ in range(DNBUF)],
        ],
    )(_decode_body)


def _decode_body(z_hbm, src_hbm, dst_hbm, out_hbm, sidx, didx, srows, drows,
                 obuf, ssem, dsem):
    c = lax.axis_index("c")
    s = lax.axis_index("s")
    w = s * NC + c
    e0 = w * E_PER_W
    pltpu.sync_copy(src_hbm.at[pl.ds(e0, E_PER_W)], sidx)
    pltpu.sync_copy(dst_hbm.at[pl.ds(e0, E_PER_W)], didx)

    def gstart(b, i):
        pltpu.async_copy(z_hbm.at[sidx.at[pl.ds(i * DCHUNK, DCHUNK)]], srows[b], ssem[b])
        pltpu.async_copy(z_hbm.at[didx.at[pl.ds(i * DCHUNK, DCHUNK)]], drows[b], dsem[b])

    def gwait(b):
        pltpu.make_async_copy(z_hbm.at[sidx.at[pl.ds(0, DCHUNK)]], srows[b], ssem[b]).wait()
        pltpu.make_async_copy(z_hbm.at[didx.at[pl.ds(0, DCHUNK)]], drows[b], dsem[b]).wait()

    lane15 = lax.iota(jnp.int32, 16) == 15

    def slot(b, i, refill):
        gwait(b)

        @pl.loop(0, DCHUNK, unroll=8)
        def _(c):
            prod = jnp.zeros((16,), jnp.float32)
            for k in range(4):
                prod = prod + srows[b][c, pl.ds(k * 16, 16)] * drows[b][c, pl.ds(k * 16, 16)]
            tot = plsc.cumsum(prod)
            prob = 1.0 / (1.0 + jnp.exp(-tot))
            eidx = jnp.full((16,), i * DCHUNK + c, jnp.int32)
            plsc.store_scatter(obuf, [eidx], prob, mask=lane15)

        if refill:
            j = i + DNBUF

            @pl.when(j < DN_CHUNKS)
            def _():
                gstart(b, j)

    for b in range(DNBUF):
        gstart(b, b)

    @pl.loop(0, DN_CHUNKS // DNBUF)
    def _(g):
        for b in range(DNBUF):
            slot(b, g * DNBUF + b, True)

    slot(0, DN_CHUNKS - 1, False)
    pltpu.sync_copy(obuf, out_hbm.at[pl.ds(w * E_PER_W, E_PER_W)])


TC_R = 512
TC_GRID = N_PAD // TC_R


def _row_spec(d):
    return pl.BlockSpec((TC_R, d), lambda i: (i, 0))


def _part_specs(d):
    return (
        pl.BlockSpec((TC_R, d), lambda i: (i, 0)),
        pl.BlockSpec((TC_R, d), lambda i: (i + TC_GRID, 0)),
    )


def _full_spec(shape):
    return pl.BlockSpec(shape, lambda i: tuple(0 for _ in shape))


def _prep_body(cnt_a, cnt_b, x, dinv_out, u1_out):
    deg = cnt_a[:, 0:1] + cnt_b[:, 0:1] + 1.0
    dinv = jnp.broadcast_to(lax.rsqrt(deg), (TC_R, 128))
    dinv_out[...] = dinv
    u1_out[...] = dinv * x[...]


def _layer1_body(s1a, s1b, u1, dinv, w1, b1, w2, u2_out):
    dv = dinv[...]
    b = dv * (s1a[...] + s1b[...] + u1[...])
    h1 = jnp.maximum(jnp.dot(b, w1[...], preferred_element_type=jnp.float32) + b1[...], 0.0)
    y2 = jnp.dot(h1, w2[...], preferred_element_type=jnp.float32)
    u2_out[...] = dv * y2


def _layer2_body(s2a, s2b, u2, dinv, b2, w3, u3_out):
    dv = dinv[...]
    h2 = jnp.maximum(dv * (s2a[...] + s2b[...] + u2[...]) + b2[...], 0.0)
    y3 = jnp.dot(h2, w3[...], preferred_element_type=jnp.float32)
    u3_out[...] = dv[:, 0:64] * y3


def _layer3_body(s3a, s3b, u3, dinv, b3, z_out):
    dv = dinv[:, 0:64]
    z_out[...] = jnp.maximum(dv * (s3a[...] + s3b[...] + u3[...]) + b3[...], 0.0)


def kernel(x, edge_index, W1, b1, W2, b2, W3, b3):
    ei = edge_index.astype(jnp.int32)
    src, dst = ei[0], ei[1]

    cnt = _make_deg_count()(dst)

    dinv, u1 = pl.pallas_call(
        _prep_body,
        grid=(TC_GRID,),
        in_specs=[*_part_specs(DEG_W), _row_spec(128)],
        out_specs=[_row_spec(128), _row_spec(128)],
        out_shape=[
            jax.ShapeDtypeStruct((N_NODES, 128), jnp.float32),
            jax.ShapeDtypeStruct((N_NODES, 128), jnp.float32),
        ],
    )(cnt, cnt, x)

    s1 = _make_segsum(128)(u1, src, dst)
    u2 = pl.pallas_call(
        _layer1_body,
        grid=(TC_GRID,),
        in_specs=[
            *_part_specs(128), _row_spec(128), _row_spec(128),
            _full_spec((128, 256)), _full_spec((1, 256)), _full_spec((256, 128)),
        ],
        out_specs=_row_spec(128),
        out_shape=jax.ShapeDtypeStruct((N_NODES, 128), jnp.float32),
    )(s1, s1, u1, dinv, W1, b1.reshape(1, -1), W2)

    s2 = _make_segsum(128)(u2, src, dst)
    u3 = pl.pallas_call(
        _layer2_body,
        grid=(TC_GRID,),
        in_specs=[
            *_part_specs(128), _row_spec(128), _row_spec(128),
            _full_spec((1, 128)), _full_spec((128, 64)),
        ],
        out_specs=_row_spec(64),
        out_shape=jax.ShapeDtypeStruct((N_NODES, 64), jnp.float32),
    )(s2, s2, u2, dinv, b2.reshape(1, -1), W3)

    s3 = _make_segsum(64)(u3, src, dst)
    z = pl.pallas_call(
        _layer3_body,
        grid=(TC_GRID,),
        in_specs=[
            *_part_specs(64), _row_spec(64), _row_spec(128), _full_spec((1, 64)),
        ],
        out_specs=_row_spec(64),
        out_shape=jax.ShapeDtypeStruct((N_NODES, 64), jnp.float32),
    )(s3, s3, u3, dinv, b3.reshape(1, -1))

    return _make_decode()(z, src, dst)


# speedup vs baseline: 22.5341x; 1.0961x over previous
import functools

import jax
import jax.numpy as jnp
from jax import lax
from jax.experimental import pallas as pl
from jax.experimental.pallas import tpu as pltpu
from jax.experimental.pallas import tpu_sc as plsc

N_NODES = 10000
N_EDGES = 320000
N_PAD = 10240

NC, NS = 2, 16
NW = NC * NS
E_PER_W = N_EDGES // NW
CHUNK = 40
N_CHUNKS = E_PER_W // CHUNK
ROWS_PER_TILE = N_PAD // NS
ZROWS = 128

_mesh = lambda: plsc.VectorSubcoreMesh(core_axis_name="c", subcore_axis_name="s")


def _zero_fill(buf, rows, cols):
    per_row = cols // 16

    @pl.loop(0, rows * per_row)
    def _(i):
        buf[i // per_row, pl.ds((i % per_row) * 16, 16)] = jnp.zeros((16,), jnp.float32)


def _zero_acc(acc, zbuf, s):
    @pl.loop(0, ROWS_PER_TILE // ZROWS)
    def _(i):
        pltpu.sync_copy(zbuf, acc.at[pl.ds(s * ROWS_PER_TILE + i * ZROWS, ZROWS)])


def _copy_out(acc, out, c, s):
    @pl.loop(0, ROWS_PER_TILE // ZROWS)
    def _(i):
        r0 = s * ROWS_PER_TILE + i * ZROWS
        pltpu.sync_copy(acc.at[pl.ds(r0, ZROWS)], out.at[pl.ds(c * N_PAD + r0, ZROWS)])


NBUF = 5
ROWS_PER_W = E_PER_W // CHUNK


@functools.lru_cache(maxsize=None)
def _make_segsum(d):
    @functools.partial(
        pl.kernel,
        out_type=jax.ShapeDtypeStruct((NC * N_PAD, d), jnp.float32),
        mesh=_mesh(),
        compiler_params=pltpu.CompilerParams(use_tc_tiling_on_sc=(d % 128 == 0)),
        scratch_types=[
            pltpu.VMEM_SHARED((N_PAD, d), jnp.float32),
            pltpu.VMEM((E_PER_W,), jnp.int32),
            pltpu.VMEM((E_PER_W,), jnp.int32),
            [pltpu.VMEM((CHUNK, d), jnp.float32) for _ in range(NBUF)],
            [pltpu.SemaphoreType.DMA for _ in range(NBUF)],
            [pltpu.SemaphoreType.DMA for _ in range(NBUF)],
        ],
    )
    def segsum(u_hbm, src_hbm, dst_hbm, out_hbm, acc, sidx, didx, rows,
               gsem, ssem):
        c = lax.axis_index("c")
        s = lax.axis_index("s")
        w = s * NC + c
        _zero_fill(rows[0], CHUNK, d)

        @pl.loop(0, ROWS_PER_TILE // CHUNK)
        def _(i):
            pltpu.sync_copy(rows[0], acc.at[pl.ds(s * ROWS_PER_TILE + i * CHUNK, CHUNK)])

        e0 = w * E_PER_W
        pltpu.sync_copy(src_hbm.at[pl.ds(e0, E_PER_W)], sidx)
        pltpu.sync_copy(dst_hbm.at[pl.ds(e0, E_PER_W)], didx)
        plsc.subcore_barrier()

        def gstart(b, i):
            pltpu.async_copy(u_hbm.at[sidx.at[pl.ds(i * CHUNK, CHUNK)]], rows[b], gsem[b])

        def gwait(b):
            pltpu.make_async_copy(u_hbm.at[sidx.at[pl.ds(0, CHUNK)]], rows[b], gsem[b]).wait()

        def sstart(b, i):
            pltpu.async_copy(rows[b], acc.at[didx.at[pl.ds(i * CHUNK, CHUNK)]], ssem[b], add=True)

        def swait(b):
            pltpu.make_async_copy(rows[b], acc.at[didx.at[pl.ds(0, CHUNK)]], ssem[b]).wait()

        for b in range(NBUF):
            gstart(b, b)

        @pl.loop(0, N_CHUNKS // NBUF)
        def _(g):
            for b in range(NBUF):
                i = g * NBUF + b
                gwait(b)
                sstart(b, i)
                pb = (b - 1) % NBUF
                j = i + NBUF - 1

                @pl.when(jnp.logical_and(i >= 1, j < N_CHUNKS))
                def _():
                    swait(pb)
                    gstart(pb, j)

        for b in range(NBUF):
            swait(b)
        plsc.subcore_barrier()
        _copy_out(acc, out_hbm, c, s)

    return segsum


DEG_W = 16


@functools.lru_cache(maxsize=None)
def _make_deg_count():
    return functools.partial(
        pl.kernel,
        out_type=jax.ShapeDtypeStruct((NC * N_PAD, DEG_W), jnp.float32),
        mesh=_mesh(),
        compiler_params=pltpu.CompilerParams(use_tc_tiling_on_sc=False),
        scratch_types=[
            pltpu.VMEM_SHARED((N_PAD, DEG_W), jnp.float32),
            pltpu.VMEM((E_PER_W,), jnp.int32),
            pltpu.VMEM((ZROWS, DEG_W), jnp.float32),
            pltpu.VMEM((CHUNK, DEG_W), jnp.float32),
            [pltpu.SemaphoreType.DMA for _ in range(NBUF)],
        ],
    )(_deg_count_body)


def _deg_count_body(dst_hbm, out_hbm, acc, didx, zbuf, ones, ssem):
    c = lax.axis_index("c")
    s = lax.axis_index("s")
    w = s * NC + c
    _zero_fill(zbuf, ZROWS, DEG_W)
    _zero_acc(acc, zbuf, s)

    per_row = DEG_W // 16

    @pl.loop(0, CHUNK * per_row)
    def _(i):
        ones[i // per_row, pl.ds((i % per_row) * 16, 16)] = jnp.ones((16,), jnp.float32)

    pltpu.sync_copy(dst_hbm.at[pl.ds(w * E_PER_W, E_PER_W)], didx)
    plsc.subcore_barrier()

    def sstart(b, i):
        pltpu.async_copy(ones, acc.at[didx.at[pl.ds(i * CHUNK, CHUNK)]], ssem[b], add=True)

    def swait(b):
        pltpu.make_async_copy(ones, acc.at[didx.at[pl.ds(0, CHUNK)]], ssem[b]).wait()

    for b in range(NBUF):
        sstart(b, b)

    @pl.loop(0, N_CHUNKS // NBUF)
    def _(g):
        for b in range(NBUF):
            j = g * NBUF + b + NBUF

            @pl.when(j < N_CHUNKS)
            def _():
                swait(b)
                sstart(b, j)

    for b in range(NBUF):
        swait(b)
    plsc.subcore_barrier()
    _copy_out(acc, out_hbm, c, s)


DCHUNK = 40
DNBUF = 2
DN_CHUNKS = E_PER_W // DCHUNK
ZROWS_PER_TILE = N_PAD // NS


@functools.lru_cache(maxsize=None)
def _make_decode():
    return functools.partial(
        pl.kernel,
        out_type=jax.ShapeDtypeStruct((N_EDGES,), jnp.float32),
        mesh=_mesh(),
        compiler_params=pltpu.CompilerParams(
            use_tc_tiling_on_sc=False, needs_layout_passes=False),
        scratch_types=[
            pltpu.VMEM_SHARED((N_PAD, 64), jnp.float32),
            [pltpu.VMEM((DCHUNK,), jnp.int32) for _ in range(DNBUF)],
            [pltpu.VMEM((DCHUNK,), jnp.int32) for _ in range(DNBUF)],
            [pltpu.VMEM((DCHUNK, 64), jnp.float32) for _ in range(DNBUF)],
            [pltpu.VMEM((DCHUNK, 64), jnp.float32) for _ in range(DNBUF)],
            pltpu.VMEM((E_PER_W,), jnp.float32),
            [pltpu.SemaphoreType.DMA for _ in range(DNBUF)],
            [pltpu.SemaphoreType.DMA for _ in range(DNBUF)],
            [pltpu.SemaphoreType.DMA for _ in range(DNBUF)],
            [pltpu.SemaphoreType.DMA for _ in range(DNBUF)],
        ],
    )(_decode_body)


def _decode_body(z_hbm, src_hbm, dst_hbm, out_hbm, zsh, sidxb, didxb,
                 srows, drows, obuf, isem, isem2, gsem, gsem2):
    c = lax.axis_index("c")
    s = lax.axis_index("s")
    w = s * NC + c
    z0 = s * ZROWS_PER_TILE
    pltpu.sync_copy(z_hbm.at[pl.ds(z0, ZROWS_PER_TILE)], zsh.at[pl.ds(z0, ZROWS_PER_TILE)])
    plsc.subcore_barrier()
    e0 = w * E_PER_W

    def istart(b, i):
        pltpu.async_copy(src_hbm.at[pl.ds(e0 + i * DCHUNK, DCHUNK)], sidxb[b], isem[b])
        pltpu.async_copy(dst_hbm.at[pl.ds(e0 + i * DCHUNK, DCHUNK)], didxb[b], isem2[b])

    def iwait(b):
        pltpu.make_async_copy(src_hbm.at[pl.ds(0, DCHUNK)], sidxb[b], isem[b]).wait()
        pltpu.make_async_copy(dst_hbm.at[pl.ds(0, DCHUNK)], didxb[b], isem2[b]).wait()

    def gstart(b):
        pltpu.async_copy(zsh.at[sidxb[b]], srows[b], gsem[b])
        pltpu.async_copy(zsh.at[didxb[b]], drows[b], gsem2[b])

    def gwait(b):
        pltpu.make_async_copy(zsh.at[sidxb[b]], srows[b], gsem[b]).wait()
        pltpu.make_async_copy(zsh.at[didxb[b]], drows[b], gsem2[b]).wait()

    lane15 = lax.iota(jnp.int32, 16) == 15

    istart(0, 0)
    istart(1, 1)
    iwait(0)
    gstart(0)

    @pl.loop(0, DN_CHUNKS // DNBUF)
    def _(g):
        for b in range(DNBUF):
            i = g * DNBUF + b
            gwait(b)

            @pl.loop(0, DCHUNK, unroll=8)
            def _(cc):
                prod = jnp.zeros((16,), jnp.float32)
                for k in range(4):
                    prod = prod + srows[b][cc, pl.ds(k * 16, 16)] * drows[b][cc, pl.ds(k * 16, 16)]
                tot = plsc.cumsum(prod)
                eidx = jnp.full((16,), i * DCHUNK + cc, jnp.int32)
                plsc.store_scatter(obuf, [eidx], tot, mask=lane15)

            @pl.when(i + 2 < DN_CHUNKS)
            def _():
                istart(b, i + 2)

            ob = (b + 1) % DNBUF

            @pl.when(i + 1 < DN_CHUNKS)
            def _():
                iwait(ob)
                gstart(ob)

    @pl.loop(0, E_PER_W // 16)
    def _(j):
        v = obuf[pl.ds(j * 16, 16)]
        obuf[pl.ds(j * 16, 16)] = 1.0 / (1.0 + jnp.exp(-v))

    pltpu.sync_copy(obuf, out_hbm.at[pl.ds(e0, E_PER_W)])


TC_R = 512
TC_GRID = N_PAD // TC_R


def _row_spec(d):
    return pl.BlockSpec((TC_R, d), lambda i: (i, 0))


def _part_specs(d):
    return (
        pl.BlockSpec((TC_R, d), lambda i: (i, 0)),
        pl.BlockSpec((TC_R, d), lambda i: (i + TC_GRID, 0)),
    )


def _full_spec(shape):
    return pl.BlockSpec(shape, lambda i: tuple(0 for _ in shape))


def _prep_body(cnt_a, cnt_b, x, dinv_out, u1_out):
    deg = cnt_a[:, 0:1] + cnt_b[:, 0:1] + 1.0
    dinv = jnp.broadcast_to(lax.rsqrt(deg), (TC_R, 128))
    dinv_out[...] = dinv
    u1_out[...] = dinv * x[...]


def _layer1_body(s1a, s1b, u1, dinv, w1, b1, w2, u2_out):
    dv = dinv[...]
    b = dv * (s1a[...] + s1b[...] + u1[...])
    h1 = jnp.maximum(jnp.dot(b, w1[...], preferred_element_type=jnp.float32) + b1[...], 0.0)
    y2 = jnp.dot(h1, w2[...], preferred_element_type=jnp.float32)
    u2_out[...] = dv * y2


def _layer2_body(s2a, s2b, u2, dinv, b2, w3, u3_out):
    dv = dinv[...]
    h2 = jnp.maximum(dv * (s2a[...] + s2b[...] + u2[...]) + b2[...], 0.0)
    y3 = jnp.dot(h2, w3[...], preferred_element_type=jnp.float32)
    u3_out[...] = dv[:, 0:64] * y3


def _layer3_body(s3a, s3b, u3, dinv, b3, z_out):
    dv = dinv[:, 0:64]
    z_out[...] = jnp.maximum(dv * (s3a[...] + s3b[...] + u3[...]) + b3[...], 0.0)


def kernel(x, edge_index, W1, b1, W2, b2, W3, b3):
    ei = edge_index.astype(jnp.int32)
    src, dst = ei[0], ei[1]

    cnt = _make_deg_count()(dst)

    dinv, u1 = pl.pallas_call(
        _prep_body,
        grid=(TC_GRID,),
        in_specs=[*_part_specs(DEG_W), _row_spec(128)],
        out_specs=[_row_spec(128), _row_spec(128)],
        out_shape=[
            jax.ShapeDtypeStruct((N_NODES, 128), jnp.float32),
            jax.ShapeDtypeStruct((N_NODES, 128), jnp.float32),
        ],
    )(cnt, cnt, x)

    s1 = _make_segsum(128)(u1, src, dst)
    u2 = pl.pallas_call(
        _layer1_body,
        grid=(TC_GRID,),
        in_specs=[
            *_part_specs(128), _row_spec(128), _row_spec(128),
            _full_spec((128, 256)), _full_spec((1, 256)), _full_spec((256, 128)),
        ],
        out_specs=_row_spec(128),
        out_shape=jax.ShapeDtypeStruct((N_NODES, 128), jnp.float32),
    )(s1, s1, u1, dinv, W1, b1.reshape(1, -1), W2)

    s2 = _make_segsum(128)(u2, src, dst)
    u3 = pl.pallas_call(
        _layer2_body,
        grid=(TC_GRID,),
        in_specs=[
            *_part_specs(128), _row_spec(128), _row_spec(128),
            _full_spec((1, 128)), _full_spec((128, 64)),
        ],
        out_specs=_row_spec(64),
        out_shape=jax.ShapeDtypeStruct((N_NODES, 64), jnp.float32),
    )(s2, s2, u2, dinv, b2.reshape(1, -1), W3)

    s3 = _make_segsum(64)(u3, src, dst)
    z = pl.pallas_call(
        _layer3_body,
        grid=(TC_GRID,),
        in_specs=[
            *_part_specs(64), _row_spec(64), _row_spec(128), _full_spec((1, 64)),
        ],
        out_specs=_row_spec(64),
        out_shape=jax.ShapeDtypeStruct((N_PAD, 64), jnp.float32),
    )(s3, s3, u3, dinv, b3.reshape(1, -1))

    return _make_decode()(z, src, dst)
